# baseline (speedup 1.0000x reference)
import dataclasses

import jax
import jax.numpy as jnp
from jax import lax
from jax.experimental import pallas as pl
from jax.experimental.pallas import tpu as pltpu
from jax.experimental.pallas import tpu_sc as plsc

B = 16384
D = 128
NC = 2
NS = 16
L = 16
NW = NC * NS
BPW = B // NW
CH = 32
NCH = BPW // CH


def _body(ci_hbm, table_hbm, out_hbm,
          cidx, buf0, buf1, res0, res1, tr,
          sem0, sem1, semo0, semo1):
    wid = lax.axis_index("s") * NC + lax.axis_index("c")
    base = wid * BPW

    iota = lax.iota(jnp.int32, L)

    pltpu.sync_copy(ci_hbm.at[pl.ds(base * 2, BPW * 2)], cidx)

    slots = ((buf0, sem0, res0, semo0), (buf1, sem1, res1, semo1))
    ROUNDS = NCH // 2

    def stream_desc(c, b):
        buf, sm = slots[b][:2]
        return pltpu.make_async_copy(
            table_hbm.at[cidx.at[pl.ds(c * (2 * CH), 2 * CH)]], buf, sm)

    def out_desc(c, b):
        _, _, res, semo = slots[b]
        return pltpu.make_async_copy(
            res, out_hbm.at[pl.ds(base + c * CH, CH)], semo)

    stream_desc(0, 0).start()
    stream_desc(1, 1).start()

    @pl.loop(0, ROUNDS)
    def _round(r):
      for b in range(2):
        buf, _, res, _ = slots[b]
        c = 2 * r + b
        stream_desc(c, b).wait()

        @pl.when(r > 0)
        def _():
            out_desc(c - 2, b).wait()

        @pl.loop(0, CH, step=L)
        def _grp(p0, vb=buf, ub=buf, res=res):
            DL = D // L

            def loads(i):
                vi = [vb[p0 + i, pl.ds(j * L, L)] for j in range(DL)]
                ui = [ub[CH + p0 + i, pl.ds(j * L, L)] for j in range(DL)]
                return vi, ui

            def dot8(vs_us):
                vs, us = vs_us
                acc0 = vs[0] * us[0]
                acc1 = vs[1] * us[1]
                for j in range(2, DL, 2):
                    acc0 = acc0 + vs[j] * us[j]
                    acc1 = acc1 + vs[j + 1] * us[j + 1]
                return acc0 + acc1

            prev = loads(0)
            for i in range(1, L):
                cur = loads(i)
                tr[i - 1, :] = dot8(prev)
                prev = cur
            tr[L - 1, :] = dot8(prev)
            tot = None
            for j in range(L):
                col = plsc.load_gather(tr, [iota, jnp.full((L,), j, jnp.int32)])
                tot = col if tot is None else tot + col
            sig = 1.0 / (1.0 + jnp.exp(-tot))
            res[pl.ds(p0, L)] = sig

        out_desc(c, b).start()

        @pl.when(r < ROUNDS - 1)
        def _():
            stream_desc(c + 2, b).start()

    out_desc(NCH - 2, 0).wait()
    out_desc(NCH - 1, 1).wait()


def kernel(pairs, kernel):
    table = kernel
    ci = pairs.reshape(NW, NCH, CH, 2)
    ci = jnp.transpose(ci, (0, 1, 3, 2)).reshape(2 * B)
    mesh = plsc.VectorSubcoreMesh(core_axis_name="c", subcore_axis_name="s")
    cp = pltpu.CompilerParams()
    if "needs_layout_passes" in pltpu.CompilerParams.__dataclass_fields__:
        cp = dataclasses.replace(cp, needs_layout_passes=False)
    k = pl.kernel(
        _body,
        out_type=jax.ShapeDtypeStruct((B,), jnp.float32),
        mesh=mesh,
        scratch_types=[
            pltpu.VMEM((2 * BPW,), jnp.int32),
            pltpu.VMEM((2 * CH, D), jnp.float32),
            pltpu.VMEM((2 * CH, D), jnp.float32),
            pltpu.VMEM((CH,), jnp.float32),
            pltpu.VMEM((CH,), jnp.float32),
            pltpu.VMEM((L, L), jnp.float32),
            pltpu.SemaphoreType.DMA,
            pltpu.SemaphoreType.DMA,
            pltpu.SemaphoreType.DMA,
            pltpu.SemaphoreType.DMA,
        ],
        compiler_params=cp,
    )
    out = k(ci, table)
    return out.reshape(B, 1)


# speedup vs baseline: 1.0610x; 1.0610x over previous
import dataclasses

import jax
import jax.numpy as jnp
from jax import lax
from jax.experimental import pallas as pl
from jax.experimental.pallas import tpu as pltpu
from jax.experimental.pallas import tpu_sc as plsc

B = 16384
D = 128
NC = 2
NS = 16
L = 16
NW = NC * NS
BPW = B // NW
CH = 64
NCH = BPW // CH


def _body(ci_hbm, table_hbm, out_hbm,
          cidx, buf0, buf1, res0, res1, tr,
          sem0, sem1, semo0, semo1):
    wid = lax.axis_index("s") * NC + lax.axis_index("c")
    base = wid * BPW

    iota = lax.iota(jnp.int32, L)

    pltpu.sync_copy(ci_hbm.at[pl.ds(base * 2, BPW * 2)], cidx)

    slots = ((buf0, sem0, res0, semo0), (buf1, sem1, res1, semo1))
    ROUNDS = NCH // 2

    def stream_desc(c, b):
        buf, sm = slots[b][:2]
        return pltpu.make_async_copy(
            table_hbm.at[cidx.at[pl.ds(c * (2 * CH), 2 * CH)]], buf, sm)

    def out_desc(c, b):
        _, _, res, semo = slots[b]
        return pltpu.make_async_copy(
            res, out_hbm.at[pl.ds(base + c * CH, CH)], semo)

    stream_desc(0, 0).start()
    stream_desc(1, 1).start()

    @pl.loop(0, ROUNDS)
    def _round(r):
      for b in range(2):
        buf, _, res, _ = slots[b]
        c = 2 * r + b
        stream_desc(c, b).wait()

        @pl.when(r > 0)
        def _():
            out_desc(c - 2, b).wait()

        @pl.loop(0, CH, step=L)
        def _grp(p0, vb=buf, ub=buf, res=res):
            DL = D // L

            def loads(i):
                vi = [vb[p0 + i, pl.ds(j * L, L)] for j in range(DL)]
                ui = [ub[CH + p0 + i, pl.ds(j * L, L)] for j in range(DL)]
                return vi, ui

            def dot8(vs_us):
                vs, us = vs_us
                acc0 = vs[0] * us[0]
                acc1 = vs[1] * us[1]
                for j in range(2, DL, 2):
                    acc0 = acc0 + vs[j] * us[j]
                    acc1 = acc1 + vs[j + 1] * us[j + 1]
                return acc0 + acc1

            prev = loads(0)
            for i in range(1, L):
                cur = loads(i)
                tr[i - 1, :] = dot8(prev)
                prev = cur
            tr[L - 1, :] = dot8(prev)
            tot = None
            for j in range(L):
                col = plsc.load_gather(tr, [iota, jnp.full((L,), j, jnp.int32)])
                tot = col if tot is None else tot + col
            sig = 1.0 / (1.0 + jnp.exp(-tot))
            res[pl.ds(p0, L)] = sig

        out_desc(c, b).start()

        @pl.when(r < ROUNDS - 1)
        def _():
            stream_desc(c + 2, b).start()

    out_desc(NCH - 2, 0).wait()
    out_desc(NCH - 1, 1).wait()


def kernel(pairs, kernel):
    table = kernel
    ci = pairs.reshape(NW, NCH, CH, 2)
    ci = jnp.transpose(ci, (0, 1, 3, 2)).reshape(2 * B)
    mesh = plsc.VectorSubcoreMesh(core_axis_name="c", subcore_axis_name="s")
    cp = pltpu.CompilerParams()
    if "needs_layout_passes" in pltpu.CompilerParams.__dataclass_fields__:
        cp = dataclasses.replace(cp, needs_layout_passes=False)
    k = pl.kernel(
        _body,
        out_type=jax.ShapeDtypeStruct((B,), jnp.float32),
        mesh=mesh,
        scratch_types=[
            pltpu.VMEM((2 * BPW,), jnp.int32),
            pltpu.VMEM((2 * CH, D), jnp.float32),
            pltpu.VMEM((2 * CH, D), jnp.float32),
            pltpu.VMEM((CH,), jnp.float32),
            pltpu.VMEM((CH,), jnp.float32),
            pltpu.VMEM((L, L), jnp.float32),
            pltpu.SemaphoreType.DMA,
            pltpu.SemaphoreType.DMA,
            pltpu.SemaphoreType.DMA,
            pltpu.SemaphoreType.DMA,
        ],
        compiler_params=cp,
    )
    out = k(ci, table)
    return out.reshape(B, 1)
